# baseline (speedup 1.0000x reference)
import jax
import jax.numpy as jnp
from jax import lax
from jax.experimental import pallas as pl
from jax.experimental.pallas import tpu as pltpu
from jax.experimental.pallas import tpu_sc as plsc

_EMBED = 32
_MAXLEN = 200
_NC = 2
_NS = 16
_NW = _NC * _NS
_CH = 128
_NBUF = 3


def _sc_body(x_hbm, tok_hbm, pos_hbm, out_hbm, idx_v, pos2_v, buf, gsem, osem):
    n_chunks_total = x_hbm.shape[0]
    nch = n_chunks_total // _NW
    rpw = nch * _CH
    wid = lax.axis_index("s") * _NC + lax.axis_index("c")
    wbase = wid * rpw

    pltpu.sync_copy(x_hbm.at[pl.ds(wid * nch, nch)], idx_v)
    pltpu.sync_copy(pos_hbm, pos2_v.at[pl.ds(0, _MAXLEN)])
    pltpu.sync_copy(pos_hbm, pos2_v.at[pl.ds(_MAXLEN, _MAXLEN)])

    pltpu.async_copy(tok_hbm.at[idx_v.at[0]], buf.at[0], gsem)

    def chunk_body(j, carry):
        b = lax.rem(j, _NBUF)
        pltpu.make_async_copy(tok_hbm.at[idx_v.at[0]], buf.at[0], gsem).wait()

        poff = lax.rem(j * _CH, _MAXLEN)

        def row_body(r, c):
            plsc.addupdate(buf.at[b, r, pl.ds(0, 16)],
                           pos2_v[poff + r, pl.ds(0, 16)])
            plsc.addupdate(buf.at[b, r, pl.ds(16, 16)],
                           pos2_v[poff + r, pl.ds(16, 16)])
            return c

        lax.fori_loop(0, _CH, row_body, 0, unroll=4)

        pltpu.async_copy(buf.at[b], out_hbm.at[pl.ds(wbase + j * _CH, _CH)],
                         osem)

        @pl.when(j >= _NBUF - 1)
        def _():
            pltpu.make_async_copy(buf.at[0], out_hbm.at[pl.ds(wbase, _CH)],
                                  osem).wait()

        @pl.when(j < nch - 1)
        def _():
            pltpu.async_copy(tok_hbm.at[idx_v.at[j + 1]],
                             buf.at[lax.rem(j + 1, _NBUF)], gsem)

        return carry

    lax.fori_loop(0, nch, chunk_body, 0)

    pltpu.make_async_copy(buf.at[0], out_hbm.at[pl.ds(wbase, _CH)], osem).wait()
    pltpu.make_async_copy(buf.at[0], out_hbm.at[pl.ds(wbase, _CH)], osem).wait()


@jax.jit
def _sc_embed(x_idx, token_table, pos_table):
    n_chunks_total = x_idx.shape[0]
    nch = n_chunks_total // _NW
    mesh = plsc.VectorSubcoreMesh(core_axis_name="c", subcore_axis_name="s")
    return pl.kernel(
        _sc_body,
        out_type=jax.ShapeDtypeStruct((n_chunks_total * _CH, _EMBED),
                                      jnp.float32),
        mesh=mesh,
        scratch_types=[
            pltpu.VMEM((nch, _CH), jnp.int32),
            pltpu.VMEM((2 * _MAXLEN, _EMBED), jnp.float32),
            pltpu.VMEM((_NBUF, _CH, _EMBED), jnp.float32),
            pltpu.SemaphoreType.DMA,
            pltpu.SemaphoreType.DMA,
        ],
        compiler_params=pltpu.CompilerParams(use_tc_tiling_on_sc=False),
    )(x_idx, token_table, pos_table)


def kernel(x, token_table, pos_table):
    batch, maxlen = x.shape
    x_idx = x.astype(jnp.int32).reshape(batch * maxlen // _CH, _CH)
    out = _sc_embed(x_idx, token_table, pos_table)
    return out.reshape(batch, maxlen, _EMBED)


# speedup vs baseline: 1.0576x; 1.0576x over previous
import jax
import jax.numpy as jnp
from jax import lax
from jax.experimental import pallas as pl
from jax.experimental.pallas import tpu as pltpu
from jax.experimental.pallas import tpu_sc as plsc

_EMBED = 32
_MAXLEN = 200
_NC = 2
_NS = 16
_NW = _NC * _NS
_CH = 128
_NBUF = 6
_DEPTH = 4


def _sc_body(x_hbm, tok_hbm, pos_hbm, out_hbm, idx_v, pos2_v, buf, gsem, osem):
    n_chunks_total = x_hbm.shape[0]
    nch = n_chunks_total // _NW
    rpw = nch * _CH
    wid = lax.axis_index("s") * _NC + lax.axis_index("c")
    wbase = wid * rpw

    pltpu.sync_copy(x_hbm.at[pl.ds(wid * nch, nch)], idx_v)
    pltpu.sync_copy(pos_hbm, pos2_v.at[pl.ds(0, _MAXLEN)])
    pltpu.sync_copy(pos_hbm, pos2_v.at[pl.ds(_MAXLEN, _MAXLEN)])

    for p in range(_DEPTH):
        pltpu.async_copy(tok_hbm.at[idx_v.at[p]], buf.at[p], gsem)

    def chunk_body(j, carry):
        b = lax.rem(j, _NBUF)
        pltpu.make_async_copy(tok_hbm.at[idx_v.at[0]], buf.at[0], gsem).wait()

        poff = lax.rem(j * _CH, _MAXLEN)

        def row_body(r, c):
            plsc.addupdate(buf.at[b, r, pl.ds(0, 16)],
                           pos2_v[poff + r, pl.ds(0, 16)])
            plsc.addupdate(buf.at[b, r, pl.ds(16, 16)],
                           pos2_v[poff + r, pl.ds(16, 16)])
            return c

        lax.fori_loop(0, _CH, row_body, 0, unroll=4)

        pltpu.async_copy(buf.at[b], out_hbm.at[pl.ds(wbase + j * _CH, _CH)],
                         osem)

        @pl.when(j >= _NBUF - _DEPTH)
        def _():
            pltpu.make_async_copy(buf.at[0], out_hbm.at[pl.ds(wbase, _CH)],
                                  osem).wait()

        @pl.when(j + _DEPTH < nch)
        def _():
            pltpu.async_copy(tok_hbm.at[idx_v.at[j + _DEPTH]],
                             buf.at[lax.rem(j + _DEPTH, _NBUF)], gsem)

        return carry

    lax.fori_loop(0, nch, chunk_body, 0)

    pltpu.make_async_copy(buf.at[0], out_hbm.at[pl.ds(wbase, _CH)], osem).wait()
    pltpu.make_async_copy(buf.at[0], out_hbm.at[pl.ds(wbase, _CH)], osem).wait()


@jax.jit
def _sc_embed(x_idx, token_table, pos_table):
    n_chunks_total = x_idx.shape[0]
    nch = n_chunks_total // _NW
    mesh = plsc.VectorSubcoreMesh(core_axis_name="c", subcore_axis_name="s")
    return pl.kernel(
        _sc_body,
        out_type=jax.ShapeDtypeStruct((n_chunks_total * _CH, _EMBED),
                                      jnp.float32),
        mesh=mesh,
        scratch_types=[
            pltpu.VMEM((nch, _CH), jnp.int32),
            pltpu.VMEM((2 * _MAXLEN, _EMBED), jnp.float32),
            pltpu.VMEM((_NBUF, _CH, _EMBED), jnp.float32),
            pltpu.SemaphoreType.DMA,
            pltpu.SemaphoreType.DMA,
        ],
        compiler_params=pltpu.CompilerParams(use_tc_tiling_on_sc=False),
    )(x_idx, token_table, pos_table)


def kernel(x, token_table, pos_table):
    batch, maxlen = x.shape
    x_idx = x.astype(jnp.int32).reshape(batch * maxlen // _CH, _CH)
    out = _sc_embed(x_idx, token_table, pos_table)
    return out.reshape(batch, maxlen, _EMBED)
